# baseline (speedup 1.0000x reference)
import functools

import numpy as np
import jax
import jax.numpy as jnp
from jax import lax
from jax.experimental import pallas as pl
from jax.experimental.pallas import tpu as pltpu
from jax.experimental.pallas import tpu_sc as plsc

VOCAB = 100000
D = 2048
SEQ = 2048
K = 4
B_BITS = 16
SEED = 42
PRIOR = 5

ROWS = 16
LANES = 128
NROW = 784
VPAD = NROW * LANES

_rng = np.random.default_rng(0)
HASH_IDX = int(_rng.integers(0, K))
np.random.seed(HASH_IDX + K * SEED)
RAND_VECS = np.random.randn(B_BITS, D).astype(np.float32)


def _threefry2x32(k0, k1, x0, x1):
    ks2 = k0 ^ k1 ^ jnp.uint32(0x1BD11BDA)
    ks = (k0, k1, ks2)
    x0 = x0 + ks[0]
    x1 = x1 + ks[1]
    rots = ((13, 15, 26, 6), (17, 29, 16, 24))
    for i in range(5):
        for r in rots[i % 2]:
            x0 = x0 + x1
            x1 = (x1 << r) | (x1 >> (32 - r))
            x1 = x1 ^ x0
        x0 = x0 + ks[(i + 1) % 3]
        x1 = x1 + ks[(i + 2) % 3] + jnp.uint32(i + 1)
    return x0, x1


@functools.cache
def _make_sc_gather():
    mesh = plsc.VectorSubcoreMesh(core_axis_name="c", subcore_axis_name="s")

    @functools.partial(
        pl.kernel,
        mesh=mesh,
        out_type=jax.ShapeDtypeStruct((ROWS, D), jnp.float32),
        scratch_types=[
            pltpu.VMEM((ROWS,), jnp.int32),
            pltpu.VMEM((ROWS, D), jnp.float32),
            pltpu.SemaphoreType.DMA,
        ],
    )
    def _sc_gather(ids_hbm, table_hbm, out_hbm, idx_v, rows_v, sem):
        wid = lax.axis_index("s") * 2 + lax.axis_index("c")

        @pl.when(wid == 0)
        def _():
            pltpu.sync_copy(ids_hbm.at[0, pl.ds(SEQ - ROWS, ROWS)], idx_v)
            pltpu.async_copy(table_hbm.at[idx_v], rows_v, sem).wait()
            pltpu.sync_copy(rows_v, out_hbm)

    return _sc_gather


SUBL = 8
LN = VOCAB // SUBL


def _tc_body(ids_ref, table_ref, logits_ref, rv_ref, out_ref, rows_v, log_v,
             sem, lsem):
    lcp = pltpu.make_async_copy(logits_ref, log_v, lsem)
    lcp.start()
    copies = []
    for j in range(PRIOR):
        idx = ids_ref[0, SEQ - PRIOR + j]
        cp = pltpu.make_async_copy(
            table_ref.at[pl.ds(idx, 1)], rows_v.at[pl.ds(j, 1)], sem)
        cp.start()
        copies.append(cp)
    for cp in copies:
        cp.wait()

    rows = rows_v[...]
    rsel = lax.broadcasted_iota(jnp.int32, (8, 1), 0) < PRIOR
    vsum = jnp.sum(jnp.where(rsel, rows, 0.0), axis=0, keepdims=True)

    mean = vsum / np.float32(PRIOR)

    proj = jnp.sum(rv_ref[...] * mean, axis=1, keepdims=True)
    powers = jnp.int32(1) << lax.broadcasted_iota(jnp.int32, (B_BITS, 1), 0)
    code = jnp.sum(jnp.where(proj > 0, powers, 0)).astype(jnp.uint32)

    hk0, hk1 = _threefry2x32(
        jnp.uint32(0), jnp.uint32(SEED), jnp.uint32(0), code)

    row = lax.broadcasted_iota(jnp.int32, (SUBL, LN), 0)
    col = lax.broadcasted_iota(jnp.int32, (SUBL, LN), 1)
    v = row * LN + col
    b0, b1 = _threefry2x32(hk0, hk1, jnp.uint32(0), v.astype(jnp.uint32))
    bits = b0 ^ b1
    f = lax.bitcast_convert_type(
        (bits >> 9) | jnp.uint32(0x3F800000), jnp.float32) - 1.0
    xi = jnp.maximum(np.float32(1e-12),
                     f * np.float32(1.0 - 1e-12) + np.float32(1e-12))

    lcp.wait()
    l1 = log_v[...]
    lp = jnp.concatenate([l1[:, i * LN:(i + 1) * LN] for i in range(SUBL)],
                         axis=0)
    m = jnp.max(lp)
    c = -jnp.log(xi) * jnp.exp(m - lp)
    cmin = jnp.min(c)
    tok = jnp.min(jnp.where(c == cmin, v, jnp.int32(VOCAB)))

    out_ref[...] = jnp.full((1, VOCAB), -100000.0, jnp.float32)
    base = pl.multiple_of((tok // 128) * 128, 128)
    lane = lax.broadcasted_iota(jnp.int32, (1, 128), 1)
    tile = jnp.where(lane == tok % 128, np.float32(100000.0),
                     np.float32(-100000.0))
    out_ref[:, pl.ds(base, 128)] = tile


def kernel(input_ids, logits, embed_table):
    ids32 = input_ids.astype(jnp.int32)

    return pl.pallas_call(
        _tc_body,
        out_shape=jax.ShapeDtypeStruct((1, VOCAB), jnp.float32),
        in_specs=[
            pl.BlockSpec(memory_space=pltpu.MemorySpace.SMEM),
            pl.BlockSpec(memory_space=pltpu.MemorySpace.HBM),
            pl.BlockSpec(memory_space=pltpu.MemorySpace.HBM),
            pl.BlockSpec(memory_space=pltpu.MemorySpace.VMEM),
        ],
        out_specs=pl.BlockSpec(memory_space=pltpu.MemorySpace.VMEM),
        scratch_shapes=[
            pltpu.VMEM((8, D), jnp.float32),
            pltpu.VMEM((1, VOCAB), jnp.float32),
            pltpu.SemaphoreType.DMA,
            pltpu.SemaphoreType.DMA,
        ],
    )(ids32, embed_table, logits, jnp.asarray(RAND_VECS))


# speedup vs baseline: 1.2976x; 1.2976x over previous
import functools

import numpy as np
import jax
import jax.numpy as jnp
from jax import lax
from jax.experimental import pallas as pl
from jax.experimental.pallas import tpu as pltpu
from jax.experimental.pallas import tpu_sc as plsc

VOCAB = 100000
D = 2048
SEQ = 2048
K = 4
B_BITS = 16
SEED = 42
PRIOR = 5

ROWS = 16
LANES = 128
NROW = 784
VPAD = NROW * LANES

_rng = np.random.default_rng(0)
HASH_IDX = int(_rng.integers(0, K))
np.random.seed(HASH_IDX + K * SEED)
RAND_VECS = np.random.randn(B_BITS, D).astype(np.float32)


def _threefry2x32(k0, k1, x0, x1):
    ks2 = k0 ^ k1 ^ jnp.uint32(0x1BD11BDA)
    ks = (k0, k1, ks2)
    x0 = x0 + ks[0]
    x1 = x1 + ks[1]
    rots = ((13, 15, 26, 6), (17, 29, 16, 24))
    for i in range(5):
        for r in rots[i % 2]:
            x0 = x0 + x1
            x1 = (x1 << r) | (x1 >> (32 - r))
            x1 = x1 ^ x0
        x0 = x0 + ks[(i + 1) % 3]
        x1 = x1 + ks[(i + 2) % 3] + jnp.uint32(i + 1)
    return x0, x1


@functools.cache
def _make_sc_gather():
    mesh = plsc.VectorSubcoreMesh(core_axis_name="c", subcore_axis_name="s")

    @functools.partial(
        pl.kernel,
        mesh=mesh,
        out_type=jax.ShapeDtypeStruct((ROWS, D), jnp.float32),
        scratch_types=[
            pltpu.VMEM((ROWS,), jnp.int32),
            pltpu.VMEM((ROWS, D), jnp.float32),
            pltpu.SemaphoreType.DMA,
        ],
    )
    def _sc_gather(ids_hbm, table_hbm, out_hbm, idx_v, rows_v, sem):
        wid = lax.axis_index("s") * 2 + lax.axis_index("c")

        @pl.when(wid == 0)
        def _():
            pltpu.sync_copy(ids_hbm.at[0, pl.ds(SEQ - ROWS, ROWS)], idx_v)
            pltpu.async_copy(table_hbm.at[idx_v], rows_v, sem).wait()
            pltpu.sync_copy(rows_v, out_hbm)

    return _sc_gather


SUBL = 8
LN = VOCAB // SUBL


def _tc_body(ids_ref, table_ref, logits_ref, rv_ref, out_ref, rows_v, sem):
    copies = []
    for j in range(PRIOR):
        idx = ids_ref[0, SEQ - PRIOR + j]
        cp = pltpu.make_async_copy(
            table_ref.at[pl.ds(idx, 1)], rows_v.at[pl.ds(j, 1)], sem)
        cp.start()
        copies.append(cp)
    for cp in copies:
        cp.wait()

    rows = rows_v[...]
    rsel = lax.broadcasted_iota(jnp.int32, (8, 1), 0) < PRIOR
    vsum = jnp.sum(jnp.where(rsel, rows, 0.0), axis=0, keepdims=True)

    mean = vsum / np.float32(PRIOR)

    proj = jnp.sum(rv_ref[...] * mean, axis=1, keepdims=True)
    powers = jnp.int32(1) << lax.broadcasted_iota(jnp.int32, (B_BITS, 1), 0)
    code = jnp.sum(jnp.where(proj > 0, powers, 0)).astype(jnp.uint32)

    hk0, hk1 = _threefry2x32(
        jnp.uint32(0), jnp.uint32(SEED), jnp.uint32(0), code)

    row = lax.broadcasted_iota(jnp.int32, (SUBL, LN), 0)
    col = lax.broadcasted_iota(jnp.int32, (SUBL, LN), 1)
    v = row * LN + col
    b0, b1 = _threefry2x32(hk0, hk1, jnp.uint32(0), v.astype(jnp.uint32))
    bits = b0 ^ b1
    f = lax.bitcast_convert_type(
        (bits >> 9) | jnp.uint32(0x3F800000), jnp.float32) - 1.0
    xi = jnp.maximum(np.float32(1e-12),
                     f * np.float32(1.0 - 1e-12) + np.float32(1e-12))

    l1 = logits_ref[...]
    lp = jnp.concatenate([l1[:, i * LN:(i + 1) * LN] for i in range(SUBL)],
                         axis=0)
    m = jnp.max(lp)
    c = -jnp.log(xi) * jnp.exp(m - lp)
    cmin = jnp.min(c)
    tok = jnp.min(jnp.where(c == cmin, v, jnp.int32(VOCAB)))

    out_ref[...] = jnp.full((1, VOCAB), -100000.0, jnp.float32)
    base = pl.multiple_of((tok // 128) * 128, 128)
    lane = lax.broadcasted_iota(jnp.int32, (1, 128), 1)
    tile = jnp.where(lane == tok % 128, np.float32(100000.0),
                     np.float32(-100000.0))
    out_ref[:, pl.ds(base, 128)] = tile


def kernel(input_ids, logits, embed_table):
    ids32 = input_ids.astype(jnp.int32)

    return pl.pallas_call(
        _tc_body,
        out_shape=jax.ShapeDtypeStruct((1, VOCAB), jnp.float32),
        in_specs=[
            pl.BlockSpec(memory_space=pltpu.MemorySpace.SMEM),
            pl.BlockSpec(memory_space=pltpu.MemorySpace.HBM),
            pl.BlockSpec(memory_space=pltpu.MemorySpace.VMEM),
            pl.BlockSpec(memory_space=pltpu.MemorySpace.VMEM),
        ],
        out_specs=pl.BlockSpec(memory_space=pltpu.MemorySpace.VMEM),
        scratch_shapes=[
            pltpu.VMEM((8, D), jnp.float32),
            pltpu.SemaphoreType.DMA,
        ],
    )(ids32, embed_table, logits, jnp.asarray(RAND_VECS))


# speedup vs baseline: 1.3125x; 1.0114x over previous
import numpy as np
import jax
import jax.numpy as jnp
from jax import lax
from jax.experimental import pallas as pl
from jax.experimental.pallas import tpu as pltpu

VOCAB = 100000
D = 2048
SEQ = 2048
K = 4
B_BITS = 16
SEED = 42
PRIOR = 5

SUBL = 8
LN = VOCAB // SUBL

_rng = np.random.default_rng(0)
HASH_IDX = int(_rng.integers(0, K))
np.random.seed(HASH_IDX + K * SEED)
RAND_VECS = np.random.randn(B_BITS, D).astype(np.float32)


def _threefry2x32(k0, k1, x0, x1):
    ks2 = k0 ^ k1 ^ jnp.uint32(0x1BD11BDA)
    ks = (k0, k1, ks2)
    x0 = x0 + ks[0]
    x1 = x1 + ks[1]
    rots = ((13, 15, 26, 6), (17, 29, 16, 24))
    for i in range(5):
        for r in rots[i % 2]:
            x0 = x0 + x1
            x1 = (x1 << r) | (x1 >> (32 - r))
            x1 = x1 ^ x0
        x0 = x0 + ks[(i + 1) % 3]
        x1 = x1 + ks[(i + 2) % 3] + jnp.uint32(i + 1)
    return x0, x1


def _tc_body(ids_ref, table_ref, logits_ref, rv_ref, out_ref, rows_v, sem):
    copies = []
    for j in range(PRIOR):
        idx = ids_ref[0, SEQ - PRIOR + j]
        cp = pltpu.make_async_copy(
            table_ref.at[pl.ds(idx, 1)], rows_v.at[pl.ds(j, 1)], sem)
        cp.start()
        copies.append(cp)
    for cp in copies:
        cp.wait()

    rows = rows_v[...]
    rsel = lax.broadcasted_iota(jnp.int32, (8, 1), 0) < PRIOR
    vsum = jnp.sum(jnp.where(rsel, rows, 0.0), axis=0, keepdims=True)
    mean = vsum / np.float32(PRIOR)

    proj = jnp.sum(rv_ref[...] * mean, axis=1, keepdims=True)
    powers = jnp.int32(1) << lax.broadcasted_iota(jnp.int32, (B_BITS, 1), 0)
    code = jnp.sum(jnp.where(proj > 0, powers, 0)).astype(jnp.uint32)

    hk0, hk1 = _threefry2x32(
        jnp.uint32(0), jnp.uint32(SEED), jnp.uint32(0), code)

    row = lax.broadcasted_iota(jnp.int32, (SUBL, LN), 0)
    col = lax.broadcasted_iota(jnp.int32, (SUBL, LN), 1)
    v = row * LN + col
    b0, b1 = _threefry2x32(hk0, hk1, jnp.uint32(0), v.astype(jnp.uint32))
    bits = b0 ^ b1
    f = lax.bitcast_convert_type(
        (bits >> 9) | jnp.uint32(0x3F800000), jnp.float32) - 1.0
    xi = jnp.maximum(np.float32(1e-12),
                     f * np.float32(1.0 - 1e-12) + np.float32(1e-12))

    l1 = logits_ref[...]
    lp = jnp.concatenate([l1[:, i * LN:(i + 1) * LN] for i in range(SUBL)],
                         axis=0)
    c = jnp.log(-jnp.log(xi)) - lp
    cmin = jnp.min(c)
    tok = jnp.min(jnp.where(c == cmin, v, jnp.int32(VOCAB)))

    out_ref[...] = jnp.full((1, VOCAB), -100000.0, jnp.float32)
    base = pl.multiple_of((tok // 128) * 128, 128)
    lane = lax.broadcasted_iota(jnp.int32, (1, 128), 1)
    tile = jnp.where(lane == tok % 128, np.float32(100000.0),
                     np.float32(-100000.0))
    out_ref[:, pl.ds(base, 128)] = tile


def kernel(input_ids, logits, embed_table):
    ids32 = input_ids.astype(jnp.int32)

    return pl.pallas_call(
        _tc_body,
        out_shape=jax.ShapeDtypeStruct((1, VOCAB), jnp.float32),
        in_specs=[
            pl.BlockSpec(memory_space=pltpu.MemorySpace.SMEM),
            pl.BlockSpec(memory_space=pltpu.MemorySpace.HBM),
            pl.BlockSpec(memory_space=pltpu.MemorySpace.VMEM),
            pl.BlockSpec(memory_space=pltpu.MemorySpace.VMEM),
        ],
        out_specs=pl.BlockSpec(memory_space=pltpu.MemorySpace.VMEM),
        scratch_shapes=[
            pltpu.VMEM((8, D), jnp.float32),
            pltpu.SemaphoreType.DMA,
        ],
    )(ids32, embed_table, logits, jnp.asarray(RAND_VECS))
